# Initial kernel scaffold; baseline (speedup 1.0000x reference)
import jax
import jax.numpy as jnp
from jax.experimental import pallas as pl


def kernel(x, edge_index, edge_attr, state, batch, bond_batch, params):
    raise NotImplementedError("write your pallas kernel here")


# speedup vs baseline: 2.7203x; 2.7203x over previous
import functools

import jax
import jax.numpy as jnp
from jax import lax
from jax.experimental import pallas as pl
from jax.experimental.pallas import tpu as pltpu
from jax.experimental.pallas import tpu_sc as plsc

F32 = jnp.float32
NC, NS = 2, 16
NW = NC * NS
SUB = 80
KSUB = 5
CH = SUB * KSUB
ET = 2000


def _mesh():
    return plsc.VectorSubcoreMesh(core_axis_name="c", subcore_axis_name="s")


def _sc_gather2(tab_s, tab_d, src, dst):
    W = tab_s.shape[1]
    E = src.shape[0]
    per_w = E // NW
    n_super = per_w // CH

    @functools.partial(
        pl.kernel,
        out_type=[jax.ShapeDtypeStruct((E, W), F32),
                  jax.ShapeDtypeStruct((E, W), F32)],
        mesh=_mesh(),
        scratch_types=[
            [pltpu.VMEM((SUB,), jnp.int32) for _ in range(KSUB)],
            [pltpu.VMEM((SUB,), jnp.int32) for _ in range(KSUB)],
            pltpu.VMEM((CH, W), F32),
            pltpu.VMEM((CH, W), F32),
            pltpu.SemaphoreType.DMA,
            pltpu.SemaphoreType.DMA,
        ],
    )
    def k(ts, td, si, di, gs, gd, sivs, divs, abuf, bbuf, s1, s2):
        wid = lax.axis_index("s") * NC + lax.axis_index("c")

        def body(c, carry):
            off = wid * per_w + c * CH
            for j in range(KSUB):
                pltpu.sync_copy(si.at[pl.ds(off + j * SUB, SUB)], sivs[j])
                pltpu.sync_copy(di.at[pl.ds(off + j * SUB, SUB)], divs[j])
            cps = []
            for j in range(KSUB):
                sl = pl.ds(j * SUB, SUB)
                cps.append(pltpu.async_copy(
                    ts.at[sivs[j]], abuf.at[sl], s1))
                cps.append(pltpu.async_copy(
                    td.at[divs[j]], bbuf.at[sl], s2))
            for cp in cps:
                cp.wait()
            pltpu.sync_copy(abuf, gs.at[pl.ds(off, CH)])
            pltpu.sync_copy(bbuf, gd.at[pl.ds(off, CH)])
            return carry

        lax.fori_loop(0, n_super, body, 0)

    return k(tab_s, tab_d, src, dst)


def _sc_scatter_add(vals, dst, n_rows, zeros):
    E, W = vals.shape
    per_w = E // NW
    n_super = per_w // CH
    rpt = n_rows // NS

    @functools.partial(
        pl.kernel,
        out_type=jax.ShapeDtypeStruct((NC * n_rows, W), F32),
        mesh=_mesh(),
        compiler_params=pltpu.CompilerParams(use_tc_tiling_on_sc=False),
        scratch_types=[
            [pltpu.VMEM((SUB,), jnp.int32) for _ in range(KSUB)],
            [pltpu.VMEM((SUB, W), F32) for _ in range(KSUB)],
            pltpu.VMEM_SHARED((n_rows, W), F32),
        ],
    )
    def k(v_h, d_h, z_h, out, idxs, rbufs, acc):
        cid = lax.axis_index("c")
        sid = lax.axis_index("s")
        wid = sid * NC + cid
        pltpu.sync_copy(z_h.at[pl.ds(sid * rpt, rpt)],
                        acc.at[pl.ds(sid * rpt, rpt)])
        plsc.subcore_barrier()

        def body(c, carry):
            off = wid * per_w + c * CH
            for j in range(KSUB):
                pltpu.sync_copy(d_h.at[pl.ds(off + j * SUB, SUB)], idxs[j])
                pltpu.sync_copy(v_h.at[pl.ds(off + j * SUB, SUB)], rbufs[j])
            for j in range(KSUB):
                pltpu.sync_copy(rbufs[j], acc.at[idxs[j]], add=True)
            return carry

        lax.fori_loop(0, n_super, body, 0)
        plsc.subcore_barrier()
        pltpu.sync_copy(acc.at[pl.ds(sid * rpt, rpt)],
                        out.at[pl.ds(cid * n_rows + sid * rpt, rpt)])

    return k(vals, dst, zeros).reshape(NC, n_rows, W)


def _tc_prep(xv, u, Ws, Wd, Wu):
    N = xv.shape[0]
    H = Ws.shape[1]
    B = u.shape[0]

    def body(x_r, u_r, ws_r, wd_r, wu_r, xs_r, xd_r, su_r):
        x = x_r[...]
        xs_r[...] = x @ ws_r[...]
        xd_r[...] = x @ wd_r[...]
        su_r[...] = u_r[...] @ wu_r[...]

    return pl.pallas_call(
        body,
        out_shape=[jax.ShapeDtypeStruct((N, H), F32),
                   jax.ShapeDtypeStruct((N, H), F32),
                   jax.ShapeDtypeStruct((B, H), F32)],
    )(xv, u, Ws, Wd, Wu)


def _tc_edge(gs, gd, ea, bb1, su, We, b1, W2, b2, skip, with_cnt):
    E, H = gs.shape
    de = ea.shape[1]
    Wo = W2.shape[1]
    G = E // ET

    def body(gs_r, gd_r, ea_r, bb_r, su_r, we_r, b1_r, w2_r, b2_r,
             e_r, bbs_r, *rest):
        i = pl.program_id(0)
        oh = (bb_r[...] == lax.broadcasted_iota(jnp.int32, (ET, 16), 1)
              ).astype(F32)
        h = (gs_r[...] + gd_r[...] + ea_r[...] @ we_r[...]
             + oh @ su_r[...] + b1_r[...])
        h = jnp.maximum(h, 0.0)
        e = h @ w2_r[...] + b2_r[...]
        if skip:
            e = e + ea_r[...]
        e_r[...] = e

        @pl.when(i == 0)
        def _():
            bbs_r[...] = jnp.zeros_like(bbs_r)
        bbs_r[...] += lax.dot_general(oh, e, (((0,), (0,)), ((), ())))

        if with_cnt:
            c_r = rest[0]

            @pl.when(i == 0)
            def _():
                c_r[...] = jnp.zeros_like(c_r)
            c_r[...] += lax.dot_general(
                oh, jnp.ones((ET, 128), F32), (((0,), (0,)), ((), ())))

    out_shape = [jax.ShapeDtypeStruct((E, Wo), F32),
                 jax.ShapeDtypeStruct((16, Wo), F32)]
    out_specs = [pl.BlockSpec((ET, Wo), lambda i: (i, 0)),
                 pl.BlockSpec((16, Wo), lambda i: (0, 0))]
    if with_cnt:
        out_shape.append(jax.ShapeDtypeStruct((16, 128), F32))
        out_specs.append(pl.BlockSpec((16, 128), lambda i: (0, 0)))

    return pl.pallas_call(
        body,
        grid=(G,),
        in_specs=[
            pl.BlockSpec((ET, H), lambda i: (i, 0)),
            pl.BlockSpec((ET, H), lambda i: (i, 0)),
            pl.BlockSpec((ET, de), lambda i: (i, 0)),
            pl.BlockSpec((ET, 1), lambda i: (i, 0)),
            pl.BlockSpec((16, H), lambda i: (0, 0)),
            pl.BlockSpec((de, H), lambda i: (0, 0)),
            pl.BlockSpec((1, H), lambda i: (0, 0)),
            pl.BlockSpec((H, Wo), lambda i: (0, 0)),
            pl.BlockSpec((1, Wo), lambda i: (0, 0)),
        ],
        out_specs=out_specs,
        out_shape=out_shape,
    )(gs, gd, ea, bb1, su, We, b1, W2, b2)


def _tc_node(xv, parts, cnt, bt1, u, bbsum, bbcnt, pv, pu, skip,
             pparts=None, pbbsum=None):
    N, dn = xv.shape
    (W1v, b1v), (W2v, b2v) = pv
    (W1u, b1u), (W2u, b2u) = pu
    Wvx, Wve, Wvu = W1v[:dn], W1v[dn:dn + 64], W1v[dn + 64:]
    Wue, Wuv, Wuu = W1u[:64], W1u[64:128], W1u[128:]
    Hv = W1v.shape[1]

    def body(*refs):
        if skip:
            (x_r, p_r, c_r, bt_r, u_r, bbs_r, bbc_r, pp_r, pbb_r,
             wvx_r, wve_r, wvu_r, b1v_r, w2v_r, b2v_r,
             wue_r, wuv_r, wuu_r, b1u_r, w2u_r, b2u_r,
             v_r, un_r) = refs
        else:
            (x_r, p_r, c_r, bt_r, u_r, bbs_r, bbc_r,
             wvx_r, wve_r, wvu_r, b1v_r, w2v_r, b2v_r,
             wue_r, wuv_r, wuu_r, b1u_r, w2u_r, b2u_r,
             v_r, un_r) = refs
        x = x_r[...]
        uu = u_r[...]
        c = c_r[0] + c_r[1]
        inv = 1.0 / jnp.maximum(c[:, :1], 1.0)
        esum = p_r[0] + p_r[1]
        bbsum_v = bbs_r[...]
        if skip:
            esum = esum - (pp_r[0] + pp_r[1])
            bbsum_v = bbsum_v - pbb_r[...]
        ev = esum * inv
        oh = (bt_r[...] == lax.broadcasted_iota(jnp.int32, (N, 16), 1)
              ).astype(F32)
        h = (x @ wvx_r[...] + ev @ wve_r[...]
             + oh @ (uu @ wvu_r[...]) + b1v_r[...])
        h = jnp.maximum(h, 0.0)
        v = h @ w2v_r[...] + b2v_r[...]
        bsum = lax.dot_general(oh, v, (((0,), (0,)), ((), ())))
        if skip:
            v = v + x
        v_r[...] = v
        bcnt = lax.dot_general(oh, jnp.ones((N, 128), F32),
                               (((0,), (0,)), ((), ())))
        bmean = bsum / jnp.maximum(bcnt[:, :64], 1.0)
        bbmean = bbsum_v / jnp.maximum(bbc_r[...][:, :64], 1.0)
        hu = (bbmean @ wue_r[...] + bmean @ wuv_r[...]
              + uu @ wuu_r[...] + b1u_r[...])
        hu = jnp.maximum(hu, 0.0)
        un = hu @ w2u_r[...] + b2u_r[...]
        if skip:
            un = un + uu
        un_r[...] = un

    args = [xv, parts, cnt, bt1, u, bbsum, bbcnt]
    if skip:
        args += [pparts, pbbsum]
    args += [Wvx, Wve, Wvu, b1v.reshape(1, Hv), W2v, b2v.reshape(1, -1),
             Wue, Wuv, Wuu, b1u.reshape(1, -1), W2u, b2u.reshape(1, -1)]
    return pl.pallas_call(
        body,
        out_shape=[jax.ShapeDtypeStruct((N, 64), F32),
                   jax.ShapeDtypeStruct((16, 64), F32)],
    )(*args)


def _tc_prep1(xv, u, Wsd, Wu):
    N = xv.shape[0]
    B = u.shape[0]
    H = Wu.shape[1]

    def body(x_r, u_r, wsd_r, wu_r, t_r, su_r):
        t_r[...] = x_r[...] @ wsd_r[...]
        su_r[...] = u_r[...] @ wu_r[...]

    return pl.pallas_call(
        body,
        out_shape=[jax.ShapeDtypeStruct((N, 128), F32),
                   jax.ShapeDtypeStruct((B, H), F32)],
    )(xv, u, Wsd, Wu)


def _tc_edge_ham(gs, gd, ea, bb1, su, We, b1, W2p, b2p, Whe, bhe):
    E = gs.shape[0]
    H = We.shape[1]
    de = ea.shape[1]
    G = E // ET

    def body(gs_r, gd_r, ea_r, bb_r, su_r, we_r, b1_r, w2_r, b2_r,
             whe_r, bhe_r, e_r, hij_r):
        oh = (bb_r[...] == lax.broadcasted_iota(jnp.int32, (ET, 16), 1)
              ).astype(F32)
        h = (gs_r[...][:, :32] + gd_r[...][:, 32:64]
             + ea_r[...] @ we_r[...] + oh @ su_r[...] + b1_r[...])
        h = jnp.maximum(h, 0.0)
        e = h @ w2_r[...] + b2_r[...]
        e_r[...] = e
        hij_r[...] = e @ whe_r[...] + bhe_r[...]

    return pl.pallas_call(
        body,
        grid=(G,),
        in_specs=[
            pl.BlockSpec((ET, 128), lambda i: (i, 0)),
            pl.BlockSpec((ET, 128), lambda i: (i, 0)),
            pl.BlockSpec((ET, de), lambda i: (i, 0)),
            pl.BlockSpec((ET, 1), lambda i: (i, 0)),
            pl.BlockSpec((16, H), lambda i: (0, 0)),
            pl.BlockSpec((de, H), lambda i: (0, 0)),
            pl.BlockSpec((1, H), lambda i: (0, 0)),
            pl.BlockSpec((H, 16), lambda i: (0, 0)),
            pl.BlockSpec((1, 16), lambda i: (0, 0)),
            pl.BlockSpec((16, 2), lambda i: (0, 0)),
            pl.BlockSpec((1, 2), lambda i: (0, 0)),
        ],
        out_specs=[pl.BlockSpec((ET, 16), lambda i: (i, 0)),
                   pl.BlockSpec((ET, 2), lambda i: (i, 0))],
        out_shape=[jax.ShapeDtypeStruct((E, 16), F32),
                   jax.ShapeDtypeStruct((E, 2), F32)],
    )(gs, gd, ea, bb1, su, We, b1, W2p, b2p, Whe, bhe)


def _tc_node_ham(xv, parts, cnt, bt1, u, pv, Whv, bhv):
    N, dn = xv.shape
    (W1v, b1v), (W2v, b2v) = pv
    Wvx, Wve, Wvu = W1v[:dn], W1v[dn:dn + 4], W1v[dn + 4:]
    Hv = W1v.shape[1]

    def body(x_r, p_r, c_r, bt_r, u_r, wvx_r, wve_r, wvu_r, b1v_r,
             w2v_r, b2v_r, whv_r, bhv_r, o_r):
        x = x_r[...]
        c = c_r[0] + c_r[1]
        inv = 1.0 / jnp.maximum(c[:, :1], 1.0)
        ev = ((p_r[0] + p_r[1]) * inv)[:, :4]
        oh = (bt_r[...] == lax.broadcasted_iota(jnp.int32, (N, 16), 1)
              ).astype(F32)
        h = (x @ wvx_r[...] + ev @ wve_r[...]
             + oh @ (u_r[...] @ wvu_r[...]) + b1v_r[...])
        h = jnp.maximum(h, 0.0)
        v = h @ w2v_r[...] + b2v_r[...]
        o_r[...] = v @ whv_r[...] + bhv_r[...]

    return pl.pallas_call(
        body,
        out_shape=jax.ShapeDtypeStruct((N, 2), F32),
    )(xv, parts, cnt, bt1, u, Wvx, Wve, Wvu, b1v.reshape(1, Hv),
      W2v, b2v.reshape(1, -1), Whv, bhv.reshape(1, -1))


def _split_phi_e(p, de, dn):
    (W1, b1), (W2, b2) = p
    We = W1[:de]
    Ws = W1[de:de + dn]
    Wd = W1[de + dn:de + 2 * dn]
    Wu = W1[de + 2 * dn:]
    return We, Ws, Wd, Wu, b1, W2, b2


def kernel(x, edge_index, edge_attr, state, batch, bond_batch, params):
    E = edge_attr.shape[0]
    N = x.shape[0]
    src2 = edge_index[0]
    dst2 = edge_index[1]
    bb1 = bond_batch.reshape(E, 1)
    bt1 = batch.reshape(N, 1)
    NPAD = 10240
    z64 = jnp.zeros((NPAD, 64), F32)
    z16 = jnp.zeros((NPAD, 16), F32)

    cnt = _sc_scatter_add(jnp.ones((E, 16), F32), dst2, NPAD, z16)[:, :N]

    xv, e, u = x, edge_attr, state
    bbcnt = None
    prev_parts = prev_bbsum = None
    for bi, name in enumerate(("block0", "block1", "block2")):
        p = params[name]
        skip = bi > 0
        de, dn = e.shape[1], xv.shape[1]
        We, Ws, Wd, Wu, b1, W2, b2 = _split_phi_e(p["phi_e"], de, dn)
        H = We.shape[1]
        xs, xd, su = _tc_prep(xv, u, Ws, Wd, Wu)
        gs, gd = _sc_gather2(xs, xd, src2, dst2)
        outs = _tc_edge(gs, gd, e, bb1, su, We, b1.reshape(1, H), W2,
                        b2.reshape(1, -1), skip, with_cnt=(bi == 0))
        if bi == 0:
            e_new, bbsum, bbcnt = outs
        else:
            e_new, bbsum = outs
        parts = _sc_scatter_add(e_new, dst2, NPAD, z64)[:, :N]
        xv, u = _tc_node(xv, parts, cnt, bt1, u, bbsum, bbcnt,
                         p["phi_v"], p["phi_u"], skip,
                         prev_parts, prev_bbsum)
        prev_parts, prev_bbsum = parts, bbsum
        e = e_new

    p = params["ham"]
    We, Ws, Wd, Wu, b1, W2, b2 = _split_phi_e(p["phi_e"], 64, 64)
    H = We.shape[1]
    W2p = jnp.zeros((H, 16), F32).at[:, :4].set(W2)
    b2p = jnp.zeros((16,), F32).at[:4].set(b2)
    (Whe, bhe), = p["head_e"]
    Whep = jnp.zeros((16, 2), F32).at[:4].set(Whe)
    (Whv, bhv), = p["head_v"]

    Wsd = jnp.zeros((64, 128), F32).at[:, :32].set(Ws).at[:, 32:64].set(Wd)
    tham, su = _tc_prep1(xv, u, Wsd, Wu)
    gs, gd = _sc_gather2(tham, tham, src2, dst2)
    e_h, ham_ij = _tc_edge_ham(gs, gd, e, bb1, su, We, b1.reshape(1, H),
                               W2p, b2p.reshape(1, 16), Whep,
                               bhe.reshape(1, 2))
    parts = _sc_scatter_add(e_h, dst2, NPAD, z16)[:, :N]
    ham_ii = _tc_node_ham(xv, parts, cnt, bt1, u, p["phi_v"], Whv, bhv)
    return ham_ii, ham_ij, edge_index
